# baseline (speedup 1.0000x reference)
import jax
import jax.numpy as jnp
from jax.experimental import pallas as pl
from jax.experimental.pallas import tpu as pltpu

_ALPHA = 1.0
_TOPK = 3
_BLOCK = 512
_NEG = -1e9


def _hyper_kernel(mask_ref, ne_ref, hy_ref, val_ref, idx_ref):
    i = pl.program_id(0)
    ne = ne_ref[...]
    hy = hy_ref[...]
    simt = jax.lax.dot_general(
        hy, ne, (((1,), (1,)), ((), ())),
        preferred_element_type=jnp.float32)
    simt = jnp.maximum(_ALPHA * simt, 0.0)
    avg = jnp.mean(mask_ref[...], axis=0)
    simt = jnp.where(avg[None, :] < 0.5, _NEG, simt)
    m = jnp.max(simt, axis=0, keepdims=True)
    e = jnp.exp(simt - m)
    soft = e / jnp.sum(e, axis=0, keepdims=True)

    h = soft.shape[0]
    row = jax.lax.broadcasted_iota(jnp.int32, soft.shape, 0)
    v = soft
    for k in range(_TOPK):
        mk = jnp.max(v, axis=0)
        ik = jnp.min(jnp.where(v == mk[None, :], row, h), axis=0)
        val_ref[k, :] = mk
        idx_ref[_TOPK + k, :] = ik
        v = jnp.where(row == ik[None, :], -1.0, v)

    node_ids = jax.lax.iota(jnp.int32, ne.shape[0]) + i * ne.shape[0]
    for k in range(_TOPK):
        idx_ref[k, :] = node_ids


def kernel(features, mask, node_embeds, hyper_embeds):
    seq_len = min(features.shape[1], node_embeds.shape[0])
    ne = node_embeds[:seq_len]
    dim = ne.shape[1]
    hnum = hyper_embeds.shape[0]
    nblk = seq_len // _BLOCK

    vals, idxs = pl.pallas_call(
        _hyper_kernel,
        grid=(nblk,),
        in_specs=[
            pl.BlockSpec((mask.shape[0], _BLOCK), lambda i: (0, i)),
            pl.BlockSpec((_BLOCK, dim), lambda i: (i, 0)),
            pl.BlockSpec((hnum, dim), lambda i: (0, 0)),
        ],
        out_specs=[
            pl.BlockSpec((8, _BLOCK), lambda i: (0, i)),
            pl.BlockSpec((8, _BLOCK), lambda i: (0, i)),
        ],
        out_shape=[
            jax.ShapeDtypeStruct((8, seq_len), jnp.float32),
            jax.ShapeDtypeStruct((8, seq_len), jnp.int32),
        ],
    )(mask, ne, hyper_embeds)

    edge_weight = vals[:_TOPK].T.reshape(-1)
    edge_index = (idxs[:2 * _TOPK].reshape(2, _TOPK, seq_len)
                  .transpose(0, 2, 1).reshape(2, -1))
    return (edge_index, edge_weight)


# speedup vs baseline: 1.6619x; 1.6619x over previous
import jax
import jax.numpy as jnp
from jax.experimental import pallas as pl
from jax.experimental.pallas import tpu as pltpu

_ALPHA = 1.0
_TOPK = 3
_BLOCK = 1024
_NEG = -1e9


def _hyper_kernel(mask_ref, ne_ref, hy_ref, val_ref, idx_ref):
    b = ne_ref.shape[0]
    ne = ne_ref[...]
    hy = hy_ref[...]
    simt = jax.lax.dot_general(
        hy, ne, (((1,), (1,)), ((), ())),
        preferred_element_type=jnp.float32)
    simt = jnp.maximum(_ALPHA * simt, 0.0)
    avg = jnp.mean(mask_ref[...], axis=0)
    simt = jnp.where(avg[None, :] < 0.5, _NEG, simt)
    m = jnp.max(simt, axis=0, keepdims=True)
    e = jnp.exp(simt - m)
    soft = e / jnp.sum(e, axis=0, keepdims=True)

    h = soft.shape[0]
    row = jax.lax.broadcasted_iota(jnp.int32, soft.shape, 0)
    v = soft
    for k in range(_TOPK):
        mk = jnp.max(v, axis=0)
        ik = jnp.min(jnp.where(v == mk[None, :], row, h), axis=0)
        val_ref[k, :] = mk
        idx_ref[k, :] = ik
        v = jnp.where(row == ik[None, :], -1.0, v)


def kernel(features, mask, node_embeds, hyper_embeds):
    seq_len = min(features.shape[1], node_embeds.shape[0])
    ne = node_embeds[:seq_len]
    dim = ne.shape[1]
    hnum = hyper_embeds.shape[0]
    nblk = seq_len // _BLOCK

    vals, idxs = pl.pallas_call(
        _hyper_kernel,
        grid=(nblk,),
        in_specs=[
            pl.BlockSpec((mask.shape[0], _BLOCK), lambda i: (0, i)),
            pl.BlockSpec((_BLOCK, dim), lambda i: (i, 0)),
            pl.BlockSpec((hnum, dim), lambda i: (0, 0)),
        ],
        out_specs=[
            pl.BlockSpec((_TOPK, _BLOCK), lambda i: (0, i)),
            pl.BlockSpec((_TOPK, _BLOCK), lambda i: (0, i)),
        ],
        out_shape=[
            jax.ShapeDtypeStruct((_TOPK, seq_len), jnp.float32),
            jax.ShapeDtypeStruct((_TOPK, seq_len), jnp.int32),
        ],
    )(mask, ne, hyper_embeds)

    n_edges = _TOPK * seq_len
    edge_weight = jax.lax.reshape(vals, (n_edges,), dimensions=(1, 0))
    cols = jax.lax.reshape(idxs, (n_edges,), dimensions=(1, 0))
    rows = jax.lax.iota(jnp.int32, n_edges) // _TOPK
    edge_index = jnp.stack([rows, cols], axis=0)
    return (edge_index, edge_weight)
